# baseline (speedup 1.0000x reference)
import functools

import jax
import jax.numpy as jnp
from jax import lax
from jax.experimental import pallas as pl
from jax.experimental.pallas import tpu as pltpu
from jax.experimental.pallas import tpu_sc as plsc

_NUM_SPEAKERS = 100000
_DIM = 64
_BATCH = 16384
_LANES = 16


@functools.cache
def _make_gather(V, D, B):
    info = plsc.get_sparse_core_info()
    NC, NS = info.num_cores, info.num_subcores
    NW = NC * NS
    assert B % NW == 0
    b_per_w = B // NW
    n_groups = b_per_w // _LANES
    mesh = plsc.VectorSubcoreMesh(core_axis_name="c", subcore_axis_name="s")

    @functools.partial(
        pl.kernel,
        mesh=mesh,
        out_type=jax.ShapeDtypeStruct((B // 2, 2 * D), jnp.float32),
        scratch_types=[
            pltpu.VMEM((b_per_w,), jnp.int32),
            pltpu.VMEM((b_per_w,), jnp.int32),
            pltpu.VMEM((b_per_w, 2 * D), jnp.float32),
            pltpu.VMEM((b_per_w // 2, 2 * D), jnp.float32),
            pltpu.SemaphoreType.DMA,
        ],
        compiler_params=pltpu.CompilerParams(needs_layout_passes=False),
    )
    def gather_kernel(table_hbm, idx_hbm, out_hbm, idx_v, pidx_v, pair_v, stag_v, sem):
        wid = lax.axis_index("s") * NC + lax.axis_index("c")
        base = wid * b_per_w
        pltpu.sync_copy(idx_hbm.at[pl.ds(base, b_per_w)], idx_v)

        iota = lax.iota(jnp.int32, _LANES)

        def compute_pidx(g, carry):
            start = pl.multiple_of(g * _LANES, _LANES)
            pidx_v[pl.ds(start, _LANES)] = idx_v[pl.ds(start, _LANES)] >> 1
            return carry

        lax.fori_loop(0, n_groups, compute_pidx, 0)
        pltpu.async_copy(table_hbm.at[pidx_v], pair_v, sem).wait()

        def extract(g, carry):
            start = pl.multiple_of(g * _LANES, _LANES)
            idxs = idx_v[pl.ds(start, _LANES)]
            off_v = (idxs & 1) << 6
            prow_v = iota + g * _LANES
            row2_v = prow_v >> 1
            off2_v = (prow_v & 1) << 6
            for c in range(D):
                vals = plsc.load_gather(pair_v, [prow_v, off_v + c])
                plsc.store_scatter(stag_v, [row2_v, off2_v + c], vals)
            return carry

        lax.fori_loop(0, n_groups, extract, 0)
        pltpu.sync_copy(stag_v, out_hbm.at[pl.ds(wid * (b_per_w // 2), b_per_w // 2)])

    return gather_kernel


@jax.jit
def kernel(spk_ids, table):
    gather = _make_gather(_NUM_SPEAKERS, _DIM, _BATCH)
    table2 = table.reshape(_NUM_SPEAKERS // 2, 2 * _DIM)
    out2 = gather(table2, spk_ids.astype(jnp.int32))
    return out2.reshape(_BATCH, _DIM)


# speedup vs baseline: 1.6049x; 1.6049x over previous
import functools

import jax
import jax.numpy as jnp
from jax import lax
from jax.experimental import pallas as pl
from jax.experimental.pallas import tpu as pltpu
from jax.experimental.pallas import tpu_sc as plsc

_NUM_SPEAKERS = 100000
_DIM = 64
_BATCH = 16384
_DPAD = 128


@functools.cache
def _make_gather(V, D, B):
    info = plsc.get_sparse_core_info()
    NC, NS = info.num_cores, info.num_subcores
    NW = NC * NS
    assert B % NW == 0
    b_per_w = B // NW
    mesh = plsc.VectorSubcoreMesh(core_axis_name="c", subcore_axis_name="s")

    @functools.partial(
        pl.kernel,
        mesh=mesh,
        out_type=jax.ShapeDtypeStruct((B, _DPAD), jnp.float32),
        scratch_types=[
            pltpu.VMEM((b_per_w,), jnp.int32),
            pltpu.VMEM((b_per_w, _DPAD), jnp.float32),
            pltpu.SemaphoreType.DMA,
        ],
    )
    def gather_kernel(table_hbm, idx_hbm, out_hbm, idx_v, rows_v, sem):
        wid = lax.axis_index("s") * NC + lax.axis_index("c")
        base = wid * b_per_w
        pltpu.sync_copy(idx_hbm.at[pl.ds(base, b_per_w)], idx_v)
        pltpu.async_copy(table_hbm.at[idx_v], rows_v, sem).wait()
        pltpu.sync_copy(rows_v, out_hbm.at[pl.ds(base, b_per_w)])

    return gather_kernel


@jax.jit
def kernel(spk_ids, table):
    gather = _make_gather(_NUM_SPEAKERS, _DIM, _BATCH)
    table_pad = jnp.pad(table, ((0, 0), (0, _DPAD - _DIM)))
    out_pad = gather(table_pad, spk_ids.astype(jnp.int32))
    return out_pad[:, :_DIM]
